# Initial kernel scaffold; baseline (speedup 1.0000x reference)
import jax
import jax.numpy as jnp
from jax.experimental import pallas as pl


def kernel(x, stem_conv1, stem_bn1, l0b0_conv1, l0b0_bn1, l0b0_conv2, l0b0_bn2, l0b0_dsconv, l0b0_dsbn, l0b1_conv1, l0b1_bn1, l0b1_conv2, l0b1_bn2, l0b2_conv1, l0b2_bn1, l0b2_conv2, l0b2_bn2, l1b0_conv1, l1b0_bn1, l1b0_conv2, l1b0_bn2, l1b0_dsconv, l1b0_dsbn, l1b1_conv1, l1b1_bn1, l1b1_conv2, l1b1_bn2, l1b2_conv1, l1b2_bn1, l1b2_conv2, l1b2_bn2, l1b3_conv1, l1b3_bn1, l1b3_conv2, l1b3_bn2, l2b0_conv1, l2b0_bn1, l2b0_conv2, l2b0_bn2, l2b0_dsconv, l2b0_dsbn, l2b1_conv1, l2b1_bn1, l2b1_conv2, l2b1_bn2, l2b2_conv1, l2b2_bn1, l2b2_conv2, l2b2_bn2, l2b3_conv1, l2b3_bn1, l2b3_conv2, l2b3_bn2, l2b4_conv1, l2b4_bn1, l2b4_conv2, l2b4_bn2, l2b5_conv1, l2b5_bn1, l2b5_conv2, l2b5_bn2, l3b0_conv1, l3b0_bn1, l3b0_conv2, l3b0_bn2, l3b0_dsconv, l3b0_dsbn, l3b1_conv1, l3b1_bn1, l3b1_conv2, l3b1_bn2, l3b2_conv1, l3b2_bn1, l3b2_conv2, l3b2_bn2, l3b3_conv1, l3b3_bn1, l3b3_conv2, l3b3_bn2, l3b4_conv1, l3b4_bn1, l3b4_conv2, l3b4_bn2, l3b5_conv1, l3b5_bn1, l3b5_conv2, l3b5_bn2, l4b0_conv1, l4b0_bn1, l4b0_conv2, l4b0_bn2, l4b0_dsconv, l4b0_dsbn, l4b1_conv1, l4b1_bn1, l4b1_conv2, l4b1_bn2, l4b2_conv1, l4b2_bn1, l4b2_conv2, l4b2_bn2):
    raise NotImplementedError("write your pallas kernel here")


# speedup vs baseline: 1.4423x; 1.4423x over previous
import functools

import jax
import jax.numpy as jnp
from jax.experimental import pallas as pl
from jax.experimental.pallas import tpu as pltpu

_BF16 = jnp.bfloat16
_VMEM_LIMIT = 100 * 1024 * 1024


def _bn_fold(bn):
    gamma, beta, mean, var = bn[0], bn[1], bn[2], bn[3]
    scale = gamma / jnp.sqrt(var + 1e-5)
    bias = beta - mean * scale
    return (scale.reshape(1, -1).astype(jnp.float32),
            bias.reshape(1, -1).astype(jnp.float32))


def _as_mat(w):
    return w.reshape(w.shape[0], w.shape[1]).T.astype(_BF16)


def _as_taps(w):
    cout, cin = w.shape[0], w.shape[1]
    return jnp.transpose(w, (2, 3, 1, 0)).reshape(9, cin, cout).astype(_BF16)


def _stem_mm_kernel(p_ref, w_ref, s_ref, b_ref, o_ref):
    acc = None
    for t in range(9):
        d = jnp.dot(p_ref[t], w_ref[t], preferred_element_type=jnp.float32)
        acc = d if acc is None else acc + d
    acc = jnp.maximum(acc * s_ref[...] + b_ref[...], 0.0)
    o_ref[...] = acc.astype(o_ref.dtype)


def _stem(x_nhwc, w, bn):
    n, h, wdt, cin = x_nhwc.shape
    cout = w.shape[0]
    xp = jnp.pad(x_nhwc, ((0, 0), (1, 1), (1, 1), (0, 0)))
    cols = [xp[:, dh:dh + h, dw:dw + wdt, :] for dh in range(3)
            for dw in range(3)]
    patches = jnp.stack([c.reshape(n * h * wdt, cin) for c in cols])
    wm = _as_taps(w)
    s, b = _bn_fold(bn)

    m = n * h * wdt
    tm = 8192
    grid = (m // tm,)
    out = pl.pallas_call(
        _stem_mm_kernel,
        grid=grid,
        in_specs=[
            pl.BlockSpec((9, tm, cin), lambda i: (0, i, 0)),
            pl.BlockSpec((9, cin, cout), lambda i: (0, 0, 0)),
            pl.BlockSpec((1, cout), lambda i: (0, 0)),
            pl.BlockSpec((1, cout), lambda i: (0, 0)),
        ],
        out_specs=pl.BlockSpec((tm, cout), lambda i: (i, 0)),
        out_shape=jax.ShapeDtypeStruct((m, cout), _BF16),
        compiler_params=pltpu.CompilerParams(
            dimension_semantics=("parallel",),
            vmem_limit_bytes=_VMEM_LIMIT),
        cost_estimate=pl.CostEstimate(
            flops=2 * m * 9 * cin * cout, transcendentals=0,
            bytes_accessed=2 * m * (9 * cin + cout)),
    )(patches, wm, s, b)
    return out.reshape(n, h, wdt, cout)


def _conv3x3_acc(t_ref, taps_ref, wp, racc, nine_offsets):
    acc = None
    for t, off in enumerate(nine_offsets):
        d = jnp.dot(t_ref[pl.ds(off, racc), :], taps_ref[t],
                    preferred_element_type=jnp.float32)
        acc = d if acc is None else acc + d
    return acc


def _stage_kernel(*refs, spec):
    wp = spec["wp"]
    racc = spec["racc"]
    rt = spec["rt"]
    strided = spec["strided"]
    nblocks = spec["nblocks"]
    has_ds0 = spec["has_ds0"]

    it = iter(refs)
    x_ref = next(it)
    wmask_ref = next(it)
    pmask_ref = next(it) if strided else None
    blk_refs = []
    for b in range(nblocks):
        n = 6 + (3 if (b == 0 and (strided or has_ds0)) else 0)
        blk_refs.append([next(it) for _ in range(n)])
    o_ref = next(it)
    t_scr = next(it)
    t4_scr = next(it) if strided else None
    bufs = [next(it), next(it)]

    t_scr[pl.ds(0, wp + 1), :] = jnp.zeros((wp + 1, t_scr.shape[1]), _BF16)
    t_scr[pl.ds(wp + 1 + racc, 2 * wp - 1), :] = jnp.zeros(
        (2 * wp - 1, t_scr.shape[1]), _BF16)

    offs = [dh * wp + dw for dh in range(3) for dw in range(3)]

    def read_input(b):
        if b == 0:
            return x_ref[0]
        return bufs[(b - 1) % 2][...]

    cur = None
    start = 0
    if strided:
        rpp = spec["rpp"]
        (w1, s1, b1, w2, s2, b2, wd, sd, bd) = blk_refs[0]
        t = jnp.dot(x_ref[0], w1[...], preferred_element_type=jnp.float32)
        t = jnp.maximum(t * s1[...] + b1[...], 0.0) * pmask_ref[...]
        t4_scr[...] = t.astype(_BF16)
        taps = []
        for dh in range(3):
            for dw in range(3):
                pidx = (dh % 2) * 2 + (dw % 2)
                off = pidx * rpp + (dh // 2) * wp + (dw // 2)
                taps.append(t4_scr[pl.ds(off, racc), :])
        patches = jnp.concatenate(taps, axis=1)
        acc = jnp.dot(patches, w2[...], preferred_element_type=jnp.float32)
        acc = acc * s2[...] + b2[...]
        res = jnp.dot(x_ref[0, pl.ds(3 * rpp, racc), :], wd[...],
                      preferred_element_type=jnp.float32)
        res = res * sd[...] + bd[...]
        res = res.astype(_BF16).astype(jnp.float32)
        cur = jnp.maximum(acc + res, 0.0)
        bufs[0][...] = cur.astype(_BF16)
        start = 1

    for b in range(start, nblocks):
        refs_b = blk_refs[b]
        ds = len(refs_b) == 9
        if ds:
            (w1, s1, b1, w2, s2, b2, wd, sd, bd) = refs_b
        else:
            (w1, s1, b1, w2, s2, b2) = refs_b
        xin = read_input(b)
        t = jnp.dot(xin, w1[...], preferred_element_type=jnp.float32)
        t = jnp.maximum(t * s1[...] + b1[...], 0.0) * wmask_ref[...]
        t_scr[pl.ds(wp + 1, racc), :] = t.astype(_BF16)
        acc = _conv3x3_acc(t_scr, w2, wp, racc, offs)
        acc = acc * s2[...] + b2[...]
        if ds:
            res = jnp.dot(xin, wd[...], preferred_element_type=jnp.float32)
            res = res * sd[...] + bd[...]
        else:
            res = xin.astype(jnp.float32)
        y = jnp.maximum(acc + res, 0.0)
        if b == nblocks - 1:
            o_ref[0] = y.astype(o_ref.dtype)
        else:
            bufs[b % 2][...] = y.astype(_BF16)


def _stage(x, blocks, h, w, strided):
    n = x.shape[0]
    if strided:
        h2, w2 = h // 2, w // 2
    else:
        h2, w2 = h, w
    wp = w2 + 2
    racc = h2 * wp
    rt = (h2 + 3) * wp
    rpp = (h2 + 2) * wp
    cin = x.shape[-1]
    cmid = blocks[0]["w1"].shape[1]
    cout = blocks[-1]["w2"].shape[-1]
    has_ds0 = "wd" in blocks[0]

    jp = jnp.arange(racc) % wp
    wmask = (jp < w2).astype(jnp.float32).reshape(racc, 1)

    in_specs = [
        pl.BlockSpec((1,) + x.shape[1:], lambda i: (i, 0, 0)),
        pl.BlockSpec((racc, 1), lambda i: (0, 0)),
    ]
    args = [x, wmask]
    if strided:
        i_idx = jnp.arange(rpp) // wp
        j_idx = jnp.arange(rpp) % wp
        pm = []
        for p in range(2):
            for q in range(2):
                a = 2 * i_idx + p
                bb = 2 * j_idx + q
                m = ((a >= 1) & (a <= h) & (bb >= 1) & (bb <= w))
                pm.append(m.astype(jnp.float32))
        pmask = jnp.concatenate(pm).reshape(4 * rpp, 1)
        in_specs.append(pl.BlockSpec((4 * rpp, 1), lambda i: (0, 0)))
        args.append(pmask)

    for blk in blocks:
        for name in ("w1", "s1", "b1", "w2", "s2", "b2", "wd", "sd", "bd"):
            if name not in blk:
                continue
            arr = blk[name]
            in_specs.append(
                pl.BlockSpec(arr.shape, lambda i, nd=arr.ndim: (0,) * nd))
            args.append(arr)

    spec = dict(wp=wp, racc=racc, rt=rt, rpp=rpp, strided=strided,
                nblocks=len(blocks), has_ds0=has_ds0)
    scratch_shapes = [pltpu.VMEM((rt, cmid), _BF16)]
    if strided:
        scratch_shapes.append(pltpu.VMEM((4 * rpp, cmid), _BF16))
    scratch_shapes += [pltpu.VMEM((racc, cout), _BF16),
                       pltpu.VMEM((racc, cout), _BF16)]

    flops = 0
    for blk in blocks:
        k1, c1 = blk["w1"].shape
        flops += 2 * racc * (k1 * c1 + 9 * cmid * blk["w2"].shape[-1])
        if "wd" in blk:
            flops += 2 * racc * k1 * blk["wd"].shape[1]
    wbytes = sum(2 * v.size for blk in blocks for v in blk.values())

    out = pl.pallas_call(
        functools.partial(_stage_kernel, spec=spec),
        grid=(n,),
        in_specs=in_specs,
        out_specs=pl.BlockSpec((1, racc, cout), lambda i: (i, 0, 0)),
        out_shape=jax.ShapeDtypeStruct((n, racc, cout), _BF16),
        scratch_shapes=scratch_shapes,
        compiler_params=pltpu.CompilerParams(
            dimension_semantics=("parallel",),
            vmem_limit_bytes=_VMEM_LIMIT),
        cost_estimate=pl.CostEstimate(
            flops=n * flops, transcendentals=0,
            bytes_accessed=2 * n * (x.shape[1] * cin + racc * cout) + wbytes),
    )(*args)
    return out


def _to_planes(x_img):
    n, h, w, c = x_img.shape
    h2, w2 = h // 2, w // 2
    wp = w2 + 2
    rpp = (h2 + 2) * wp
    xp = jnp.pad(x_img, ((0, 0), (1, 1), (1, 3), (0, 0)))
    pl4 = xp.reshape(n, h2 + 1, 2, wp, 2, c)
    pl4 = jnp.transpose(pl4, (0, 2, 4, 1, 3, 5))
    pl4 = pl4.reshape(n, 4, (h2 + 1) * wp, c)
    pl4 = jnp.pad(pl4, ((0, 0), (0, 0), (0, rpp - (h2 + 1) * wp), (0, 0)))
    return pl4.reshape(n, 4 * rpp, c)


def _from_acc(out, n, h, w):
    c = out.shape[-1]
    return out.reshape(n, h, w + 2, c)[:, :, :w, :]


def _mk_blocks(params, strided_first=False):
    blocks = []
    for bi, p in enumerate(params):
        if bi == 0 and strided_first:
            w2 = jnp.transpose(p[2], (2, 3, 1, 0)).reshape(
                9 * p[2].shape[1], p[2].shape[0]).astype(_BF16)
        else:
            w2 = _as_taps(p[2])
        blk = {"w1": _as_mat(p[0]), "w2": w2}
        blk["s1"], blk["b1"] = _bn_fold(p[1])
        blk["s2"], blk["b2"] = _bn_fold(p[3])
        if len(p) > 4:
            blk["wd"] = _as_mat(p[4])
            blk["sd"], blk["bd"] = _bn_fold(p[5])
        blocks.append(blk)
    return blocks


def kernel(x, stem_conv1, stem_bn1, l0b0_conv1, l0b0_bn1, l0b0_conv2, l0b0_bn2, l0b0_dsconv, l0b0_dsbn, l0b1_conv1, l0b1_bn1, l0b1_conv2, l0b1_bn2, l0b2_conv1, l0b2_bn1, l0b2_conv2, l0b2_bn2, l1b0_conv1, l1b0_bn1, l1b0_conv2, l1b0_bn2, l1b0_dsconv, l1b0_dsbn, l1b1_conv1, l1b1_bn1, l1b1_conv2, l1b1_bn2, l1b2_conv1, l1b2_bn1, l1b2_conv2, l1b2_bn2, l1b3_conv1, l1b3_bn1, l1b3_conv2, l1b3_bn2, l2b0_conv1, l2b0_bn1, l2b0_conv2, l2b0_bn2, l2b0_dsconv, l2b0_dsbn, l2b1_conv1, l2b1_bn1, l2b1_conv2, l2b1_bn2, l2b2_conv1, l2b2_bn1, l2b2_conv2, l2b2_bn2, l2b3_conv1, l2b3_bn1, l2b3_conv2, l2b3_bn2, l2b4_conv1, l2b4_bn1, l2b4_conv2, l2b4_bn2, l2b5_conv1, l2b5_bn1, l2b5_conv2, l2b5_bn2, l3b0_conv1, l3b0_bn1, l3b0_conv2, l3b0_bn2, l3b0_dsconv, l3b0_dsbn, l3b1_conv1, l3b1_bn1, l3b1_conv2, l3b1_bn2, l3b2_conv1, l3b2_bn1, l3b2_conv2, l3b2_bn2, l3b3_conv1, l3b3_bn1, l3b3_conv2, l3b3_bn2, l3b4_conv1, l3b4_bn1, l3b4_conv2, l3b4_bn2, l3b5_conv1, l3b5_bn1, l3b5_conv2, l3b5_bn2, l4b0_conv1, l4b0_bn1, l4b0_conv2, l4b0_bn2, l4b0_dsconv, l4b0_dsbn, l4b1_conv1, l4b1_bn1, l4b1_conv2, l4b1_bn2, l4b2_conv1, l4b2_bn1, l4b2_conv2, l4b2_bn2):
    n = x.shape[0]
    x_nhwc = jnp.transpose(x, (0, 2, 3, 1)).astype(_BF16)

    t = _stem(x_nhwc, stem_conv1, stem_bn1)

    s0 = _mk_blocks([
        (l0b0_conv1, l0b0_bn1, l0b0_conv2, l0b0_bn2, l0b0_dsconv, l0b0_dsbn),
        (l0b1_conv1, l0b1_bn1, l0b1_conv2, l0b1_bn2),
        (l0b2_conv1, l0b2_bn1, l0b2_conv2, l0b2_bn2)], strided_first=True)
    a = _stage(_to_planes(t), s0, 128, 128, strided=True)

    s1 = _mk_blocks([
        (l1b0_conv1, l1b0_bn1, l1b0_conv2, l1b0_bn2, l1b0_dsconv, l1b0_dsbn),
        (l1b1_conv1, l1b1_bn1, l1b1_conv2, l1b1_bn2),
        (l1b2_conv1, l1b2_bn1, l1b2_conv2, l1b2_bn2),
        (l1b3_conv1, l1b3_bn1, l1b3_conv2, l1b3_bn2)])
    a = _stage(a, s1, 64, 64, strided=False)

    s2 = _mk_blocks([
        (l2b0_conv1, l2b0_bn1, l2b0_conv2, l2b0_bn2, l2b0_dsconv, l2b0_dsbn),
        (l2b1_conv1, l2b1_bn1, l2b1_conv2, l2b1_bn2),
        (l2b2_conv1, l2b2_bn1, l2b2_conv2, l2b2_bn2),
        (l2b3_conv1, l2b3_bn1, l2b3_conv2, l2b3_bn2),
        (l2b4_conv1, l2b4_bn1, l2b4_conv2, l2b4_bn2),
        (l2b5_conv1, l2b5_bn1, l2b5_conv2, l2b5_bn2)], strided_first=True)
    a = _stage(_to_planes(_from_acc(a, n, 64, 64)), s2, 64, 64, strided=True)

    s3 = _mk_blocks([
        (l3b0_conv1, l3b0_bn1, l3b0_conv2, l3b0_bn2, l3b0_dsconv, l3b0_dsbn),
        (l3b1_conv1, l3b1_bn1, l3b1_conv2, l3b1_bn2),
        (l3b2_conv1, l3b2_bn1, l3b2_conv2, l3b2_bn2),
        (l3b3_conv1, l3b3_bn1, l3b3_conv2, l3b3_bn2),
        (l3b4_conv1, l3b4_bn1, l3b4_conv2, l3b4_bn2),
        (l3b5_conv1, l3b5_bn1, l3b5_conv2, l3b5_bn2)])
    a = _stage(a, s3, 32, 32, strided=False)

    s4 = _mk_blocks([
        (l4b0_conv1, l4b0_bn1, l4b0_conv2, l4b0_bn2, l4b0_dsconv, l4b0_dsbn),
        (l4b1_conv1, l4b1_bn1, l4b1_conv2, l4b1_bn2),
        (l4b2_conv1, l4b2_bn1, l4b2_conv2, l4b2_bn2)])
    a = _stage(a, s4, 32, 32, strided=False)

    out = _from_acc(a, n, 32, 32)
    return jnp.transpose(out, (0, 3, 1, 2)).astype(jnp.float32)
